# baseline (speedup 1.0000x reference)
import functools
import jax
import jax.numpy as jnp
from jax import lax
from jax.experimental import pallas as pl
from jax.experimental.pallas import tpu as pltpu
from jax.experimental.pallas import tpu_sc as plsc

_N_PATCHES = 16
_CH = 8
_NBUF = 2


@functools.lru_cache(maxsize=None)
def _make_sc_kernel(R, D, S):
    info = plsc.get_sparse_core_info()
    NC, NS, L = info.num_cores, info.num_subcores, info.num_lanes
    NW = NC * NS
    rows_w = R // NW
    rpp = S // _N_PATCHES
    ppw = rows_w // rpp
    wpb = S // rows_w
    nchunks = rows_w // _CH
    cpp = rpp // _CH
    cols = D // L
    csz = _CH * D

    mesh = plsc.VectorSubcoreMesh(core_axis_name="c", subcore_axis_name="s")

    @functools.partial(
        pl.kernel,
        out_type=jax.ShapeDtypeStruct((R * D,), jnp.float32),
        mesh=mesh,
        scratch_types=[
            pltpu.VMEM((ppw * D,), jnp.float32),
            [pltpu.VMEM((csz,), jnp.float32)] * _NBUF,
            [pltpu.VMEM((csz,), jnp.float32)] * _NBUF,
            [pltpu.SemaphoreType.DMA] * _NBUF,
            [pltpu.SemaphoreType.DMA] * _NBUF,
        ],
    )
    def sc_k(vf_hbm, w_hbm, out_hbm, w_buf, ins, outs, sins, souts):
        cid = lax.axis_index("c")
        sid = lax.axis_index("s")
        wid = sid * NC + cid
        row0 = wid * rows_w
        p0 = (wid % wpb) * ppw
        pltpu.sync_copy(w_hbm.at[pl.ds(p0 * D, ppw * D)], w_buf)

        def issue_gather(g, b):
            base = (row0 + g * _CH) * D
            pltpu.async_copy(vf_hbm.at[pl.ds(base, csz)], ins[b], sins[b])

        def issue_scatter(g, b):
            base = (row0 + g * _CH) * D
            pltpu.async_copy(outs[b], out_hbm.at[pl.ds(base, csz)], souts[b])

        def wait_in(b):
            pltpu.make_async_copy(vf_hbm.at[pl.ds(0, csz)], ins[b], sins[b]).wait()

        def wait_out(b):
            pltpu.make_async_copy(outs[b], out_hbm.at[pl.ds(0, csz)], souts[b]).wait()

        for b in range(_NBUF):
            issue_gather(b, b)

        def pair(t, carry):
            for b in range(_NBUF):
                g = t * _NBUF + b
                wait_in(b)
                @pl.when(g >= _NBUF)
                def _():
                    wait_out(b)

                woff = (g // cpp) * D

                def col(c, cc):
                    wv = w_buf[pl.ds(woff + c * L, L)]
                    for r in range(_CH):
                        off = r * D + c * L
                        outs[b][pl.ds(off, L)] = ins[b][pl.ds(off, L)] + wv
                    return cc

                lax.fori_loop(0, cols, col, 0)
                issue_scatter(g, b)

                @pl.when(g + _NBUF < nchunks)
                def _():
                    issue_gather(g + _NBUF, b)

            return carry

        lax.fori_loop(0, nchunks // _NBUF, pair, 0)
        for b in range(_NBUF):
            wait_out(b)

    return sc_k


def kernel(vision_features, W):
    B, S, D = vision_features.shape
    R = B * S
    vf = vision_features.reshape(R * D)
    w_flat = W.reshape(-1)
    sc_k = _make_sc_kernel(R, D, S)
    out = sc_k(vf, w_flat)
    return out.reshape(B, S, D)


# speedup vs baseline: 1.3641x; 1.3641x over previous
import functools
import jax
import jax.numpy as jnp
from jax import lax
from jax.experimental import pallas as pl
from jax.experimental.pallas import tpu as pltpu
from jax.experimental.pallas import tpu_sc as plsc

_N_PATCHES = 16
_CH = 16


@functools.lru_cache(maxsize=None)
def _make_sc_kernel(R, D, S):
    info = plsc.get_sparse_core_info()
    NC, NS, L = info.num_cores, info.num_subcores, info.num_lanes
    NW = NC * NS
    rows_w = R // NW
    rpp = S // _N_PATCHES
    ppw = rows_w // rpp
    wpb = S // rows_w
    nchunks = rows_w // _CH
    cpp = rpp // _CH
    cols = D // L
    csz = _CH * D

    mesh = plsc.VectorSubcoreMesh(core_axis_name="c", subcore_axis_name="s")

    @functools.partial(
        pl.kernel,
        out_type=jax.ShapeDtypeStruct((R * D,), jnp.float32),
        mesh=mesh,
        scratch_types=[
            pltpu.VMEM((ppw * D,), jnp.float32),
            pltpu.VMEM((csz,), jnp.float32),
        ],
    )
    def sc_k(vf_hbm, w_hbm, out_hbm, w_buf, buf):
        cid = lax.axis_index("c")
        sid = lax.axis_index("s")
        wid = sid * NC + cid
        row0 = wid * rows_w
        p0 = (wid % wpb) * ppw
        pltpu.sync_copy(w_hbm.at[pl.ds(p0 * D, ppw * D)], w_buf)

        def chunk(i, carry):
            base = (row0 + i * _CH) * D
            woff = (i // cpp) * D
            pltpu.sync_copy(vf_hbm.at[pl.ds(base, csz)], buf)

            def col(c, cc):
                wv = w_buf[pl.ds(woff + c * L, L)]
                for r in range(_CH):
                    off = r * D + c * L
                    plsc.addupdate(buf.at[pl.ds(off, L)], wv)
                return cc

            lax.fori_loop(0, cols, col, 0)
            pltpu.sync_copy(buf, out_hbm.at[pl.ds(base, csz)])
            return carry

        lax.fori_loop(0, nchunks, chunk, 0)

    return sc_k


def kernel(vision_features, W):
    B, S, D = vision_features.shape
    R = B * S
    vf = vision_features.reshape(R * D)
    w_flat = W.reshape(-1)
    sc_k = _make_sc_kernel(R, D, S)
    out = sc_k(vf, w_flat)
    return out.reshape(B, S, D)


# speedup vs baseline: 1.6149x; 1.1839x over previous
import functools
import jax
import jax.numpy as jnp
from jax import lax
from jax.experimental import pallas as pl
from jax.experimental.pallas import tpu as pltpu
from jax.experimental.pallas import tpu_sc as plsc

_N_PATCHES = 16
_CH = 8
_NBUF = 4


@functools.lru_cache(maxsize=None)
def _make_sc_kernel(R, D, S):
    info = plsc.get_sparse_core_info()
    NC, NS, L = info.num_cores, info.num_subcores, info.num_lanes
    NW = NC * NS
    rows_w = R // NW
    rpp = S // _N_PATCHES
    ppw = rows_w // rpp
    wpb = S // rows_w
    nchunks = rows_w // _CH
    cpp = rpp // _CH
    cols = D // L
    csz = _CH * D

    mesh = plsc.VectorSubcoreMesh(core_axis_name="c", subcore_axis_name="s")

    @functools.partial(
        pl.kernel,
        out_type=jax.ShapeDtypeStruct((R * D,), jnp.float32),
        mesh=mesh,
        scratch_types=[
            pltpu.VMEM((ppw * D,), jnp.float32),
            [pltpu.VMEM((csz,), jnp.float32)] * _NBUF,
            [pltpu.SemaphoreType.DMA] * _NBUF,
            [pltpu.SemaphoreType.DMA] * _NBUF,
        ],
    )
    def sc_k(vf_hbm, w_hbm, out_hbm, w_buf, bufs, sins, souts):
        cid = lax.axis_index("c")
        sid = lax.axis_index("s")
        wid = sid * NC + cid
        row0 = wid * rows_w
        p0 = (wid % wpb) * ppw
        pltpu.sync_copy(w_hbm.at[pl.ds(p0 * D, ppw * D)], w_buf)

        def issue_gather(g, b):
            base = (row0 + g * _CH) * D
            pltpu.async_copy(vf_hbm.at[pl.ds(base, csz)], bufs[b], sins[b])

        def issue_scatter(g, b):
            base = (row0 + g * _CH) * D
            pltpu.async_copy(bufs[b], out_hbm.at[pl.ds(base, csz)], souts[b])

        def wait_in(b):
            pltpu.make_async_copy(vf_hbm.at[pl.ds(0, csz)], bufs[b], sins[b]).wait()

        def wait_out(b):
            pltpu.make_async_copy(bufs[b], out_hbm.at[pl.ds(0, csz)], souts[b]).wait()

        issue_gather(0, 0)
        issue_gather(1, 1)

        def quad(t, carry):
            for b in range(_NBUF):
                j = t * _NBUF + b
                wait_in(b)

                woff = (j // cpp) * D

                def col(c, cc):
                    wv = w_buf[pl.ds(woff + c * L, L)]
                    for r in range(_CH):
                        off = r * D + c * L
                        plsc.addupdate(bufs[b].at[pl.ds(off, L)], wv)
                    return cc

                lax.fori_loop(0, cols, col, 0)
                issue_scatter(j, b)

                bn = (b + 2) % _NBUF

                @pl.when(j >= 2)
                def _():
                    wait_out(bn)

                @pl.when(j + 2 < nchunks)
                def _():
                    issue_gather(j + 2, bn)

            return carry

        lax.fori_loop(0, nchunks // _NBUF, quad, 0)
        wait_out((nchunks - 2) % _NBUF)
        wait_out((nchunks - 1) % _NBUF)

    return sc_k


def kernel(vision_features, W):
    B, S, D = vision_features.shape
    R = B * S
    vf = vision_features.reshape(R * D)
    w_flat = W.reshape(-1)
    sc_k = _make_sc_kernel(R, D, S)
    out = sc_k(vf, w_flat)
    return out.reshape(B, S, D)


# speedup vs baseline: 4.5733x; 2.8319x over previous
import functools
import jax
import jax.numpy as jnp
from jax import lax
from jax.experimental import pallas as pl
from jax.experimental.pallas import tpu as pltpu
from jax.experimental.pallas import tpu_sc as plsc

_N_PATCHES = 16
_CH = 8
_NBUF = 4


@functools.lru_cache(maxsize=None)
def _make_sc_kernel(R, D, S):
    info = plsc.get_sparse_core_info()
    NC, NS, L = info.num_cores, info.num_subcores, info.num_lanes
    NW = NC * NS
    rows_w = R // NW
    rpp = S // _N_PATCHES
    ppw = rows_w // rpp
    wpb = S // rows_w
    nchunks = rows_w // _CH
    cpp = rpp // _CH
    cols = D // L

    mesh = plsc.VectorSubcoreMesh(core_axis_name="c", subcore_axis_name="s")

    @functools.partial(
        pl.kernel,
        out_type=jax.ShapeDtypeStruct((R, D), jnp.float32),
        mesh=mesh,
        scratch_types=[
            pltpu.VMEM((_N_PATCHES, D), jnp.float32),
            [pltpu.VMEM((_CH, D), jnp.float32)] * _NBUF,
            [pltpu.SemaphoreType.DMA] * _NBUF,
            [pltpu.SemaphoreType.DMA] * _NBUF,
        ],
        compiler_params=pltpu.CompilerParams(use_tc_tiling_on_sc=True),
    )
    def sc_k(vf_hbm, w_hbm, out_hbm, w_buf, bufs, sins, souts):
        cid = lax.axis_index("c")
        sid = lax.axis_index("s")
        wid = sid * NC + cid
        row0 = wid * rows_w
        p0 = (wid % wpb) * ppw
        pltpu.sync_copy(w_hbm.at[pl.ds(0, _N_PATCHES)], w_buf)

        def issue_gather(g, b):
            rs = row0 + g * _CH
            pltpu.async_copy(vf_hbm.at[pl.ds(rs, _CH)], bufs[b], sins[b])

        def issue_scatter(g, b):
            rs = row0 + g * _CH
            pltpu.async_copy(bufs[b], out_hbm.at[pl.ds(rs, _CH)], souts[b])

        def wait_in(b):
            pltpu.make_async_copy(vf_hbm.at[pl.ds(0, _CH)], bufs[b], sins[b]).wait()

        def wait_out(b):
            pltpu.make_async_copy(bufs[b], out_hbm.at[pl.ds(0, _CH)], souts[b]).wait()

        issue_gather(0, 0)
        issue_gather(1, 1)

        def quad(t, carry):
            for b in range(_NBUF):
                j = t * _NBUF + b
                wait_in(b)

                p = p0 + j // cpp

                def col(c, cc):
                    wv = w_buf[p, pl.ds(c * L, L)]
                    for r in range(_CH):
                        plsc.addupdate(bufs[b].at[r, pl.ds(c * L, L)], wv)
                    return cc

                lax.fori_loop(0, cols, col, 0)
                issue_scatter(j, b)

                bn = (b + 2) % _NBUF

                @pl.when(j >= 2)
                def _():
                    wait_out(bn)

                @pl.when(j + 2 < nchunks)
                def _():
                    issue_gather(j + 2, bn)

            return carry

        lax.fori_loop(0, nchunks // _NBUF, quad, 0)
        wait_out((nchunks - 2) % _NBUF)
        wait_out((nchunks - 1) % _NBUF)

    return sc_k


def kernel(vision_features, W):
    B, S, D = vision_features.shape
    R = B * S
    vf = vision_features.reshape(R, D)
    sc_k = _make_sc_kernel(R, D, S)
    out = sc_k(vf, W)
    return out.reshape(B, S, D)


# speedup vs baseline: 4.7517x; 1.0390x over previous
import functools
import jax
import jax.numpy as jnp
from jax import lax
from jax.experimental import pallas as pl
from jax.experimental.pallas import tpu as pltpu
from jax.experimental.pallas import tpu_sc as plsc

_N_PATCHES = 16
_CH = 8
_NBUF = 4


@functools.lru_cache(maxsize=None)
def _make_sc_kernel(R, D, S):
    info = plsc.get_sparse_core_info()
    NC, NS, L = info.num_cores, info.num_subcores, info.num_lanes
    NW = NC * NS
    rows_w = R // NW
    rpp = S // _N_PATCHES
    ppw = rows_w // rpp
    wpb = S // rows_w
    nchunks = rows_w // _CH
    cpp = rpp // _CH
    cols = D // L

    mesh = plsc.VectorSubcoreMesh(core_axis_name="c", subcore_axis_name="s")

    @functools.partial(
        pl.kernel,
        out_type=jax.ShapeDtypeStruct((R, D), jnp.float32),
        mesh=mesh,
        scratch_types=[
            pltpu.VMEM((_N_PATCHES, D), jnp.float32),
            [pltpu.VMEM((_CH, D), jnp.float32)] * _NBUF,
            [pltpu.SemaphoreType.DMA] * _NBUF,
            [pltpu.SemaphoreType.DMA] * _NBUF,
        ],
        compiler_params=pltpu.CompilerParams(use_tc_tiling_on_sc=True),
    )
    def sc_k(vf_hbm, w_hbm, out_hbm, w_buf, bufs, sins, souts):
        cid = lax.axis_index("c")
        sid = lax.axis_index("s")
        wid = sid * NC + cid
        row0 = wid * rows_w
        p0 = (wid % wpb) * ppw
        pltpu.sync_copy(w_hbm.at[pl.ds(0, _N_PATCHES)], w_buf)

        def issue_gather(g, b):
            rs = row0 + g * _CH
            pltpu.async_copy(vf_hbm.at[pl.ds(rs, _CH)], bufs[b], sins[b])

        def issue_scatter(g, b):
            rs = row0 + g * _CH
            pltpu.async_copy(bufs[b], out_hbm.at[pl.ds(rs, _CH)], souts[b])

        def wait_in(b):
            pltpu.make_async_copy(vf_hbm.at[pl.ds(0, _CH)], bufs[b], sins[b]).wait()

        def wait_out(b):
            pltpu.make_async_copy(bufs[b], out_hbm.at[pl.ds(0, _CH)], souts[b]).wait()

        issue_gather(0, 0)
        issue_gather(1, 1)

        def quad(t, carry):
            for b in range(_NBUF):
                j = t * _NBUF + b
                wait_in(b)

                p = p0 + j // cpp

                @plsc.parallel_loop(0, cols, 1, unroll=4)
                def col(c):
                    wv = w_buf[p, pl.ds(c * L, L)]
                    for r in range(_CH):
                        plsc.addupdate(bufs[b].at[r, pl.ds(c * L, L)], wv)
                issue_scatter(j, b)

                bn = (b + 2) % _NBUF

                @pl.when(j >= 2)
                def _():
                    wait_out(bn)

                @pl.when(j + 2 < nchunks)
                def _():
                    issue_gather(j + 2, bn)

            return carry

        lax.fori_loop(0, nchunks // _NBUF, quad, 0)
        wait_out((nchunks - 2) % _NBUF)
        wait_out((nchunks - 1) % _NBUF)

    return sc_k


def kernel(vision_features, W):
    B, S, D = vision_features.shape
    R = B * S
    vf = vision_features.reshape(R, D)
    sc_k = _make_sc_kernel(R, D, S)
    out = sc_k(vf, W)
    return out.reshape(B, S, D)
